# baseline (speedup 1.0000x reference)
import functools

import jax
import jax.numpy as jnp
from jax import lax
from jax.experimental import pallas as pl
from jax.experimental.pallas import tpu as pltpu
from jax.experimental.pallas import tpu_sc as plsc

BATCH_SIZE = 16384
DIM = 64
NUM_CORES = 2
NUM_SUBCORES = 16
NUM_WORKERS = NUM_CORES * NUM_SUBCORES
BPW = BATCH_SIZE // NUM_WORKERS
CHUNK = 128
N_CHUNKS = BPW // CHUNK
LANES = 16
N_GROUPS = CHUNK // LANES


def _dots_body(jt, hv, uemb, iemb, out_up, out_un,
               jt_v, hv_v, ru, rp, rn, acc_up, acc_un, sem):
    wid = lax.axis_index("s") * NUM_CORES + lax.axis_index("c")
    base = wid * BPW
    pltpu.sync_copy(jt.at[wid], jt_v)
    pltpu.sync_copy(hv.at[wid], hv_v)

    def chunk_step(c, carry):
        h1 = pltpu.async_copy(uemb.at[jt_v.at[0, c]], ru, sem)
        h2 = pltpu.async_copy(iemb.at[jt_v.at[1, c]], rp, sem)
        h3 = pltpu.async_copy(iemb.at[jt_v.at[2, c]], rn, sem)
        h1.wait()
        h2.wait()
        h3.wait()

        def group_step(g, carry2):
            rows = lax.iota(jnp.int32, LANES) + g * LANES
            cu = hv_v[0, c, pl.ds(g * LANES, LANES)] * DIM
            cp = hv_v[1, c, pl.ds(g * LANES, LANES)] * DIM
            cn = hv_v[2, c, pl.ds(g * LANES, LANES)] * DIM
            aup = jnp.zeros((LANES,), jnp.float32)
            aun = jnp.zeros((LANES,), jnp.float32)
            for d in range(DIM):
                du = plsc.load_gather(ru, [rows, cu + d])
                dp = plsc.load_gather(rp, [rows, cp + d])
                dn = plsc.load_gather(rn, [rows, cn + d])
                aup = aup + du * dp
                aun = aun + du * dn
            off = c * CHUNK + g * LANES
            acc_up[pl.ds(off, LANES)] = aup
            acc_un[pl.ds(off, LANES)] = aun
            return carry2

        lax.fori_loop(0, N_GROUPS, group_step, 0)
        return carry

    lax.fori_loop(0, N_CHUNKS, chunk_step, 0)
    pltpu.sync_copy(acc_up, out_up.at[pl.ds(base, BPW)])
    pltpu.sync_copy(acc_un, out_un.at[pl.ds(base, BPW)])


_DOTS = jax.ShapeDtypeStruct((BATCH_SIZE,), jnp.float32)


@functools.cache
def _sc_dots():
    return functools.partial(
        pl.kernel,
        mesh=plsc.VectorSubcoreMesh(core_axis_name="c", subcore_axis_name="s"),
        out_type=(_DOTS, _DOTS),
        scratch_types=[
            pltpu.VMEM((3, N_CHUNKS, CHUNK), jnp.int32),
            pltpu.VMEM((3, N_CHUNKS, CHUNK), jnp.int32),
            pltpu.VMEM((CHUNK, 2 * DIM), jnp.float32),
            pltpu.VMEM((CHUNK, 2 * DIM), jnp.float32),
            pltpu.VMEM((CHUNK, 2 * DIM), jnp.float32),
            pltpu.VMEM((BPW,), jnp.float32),
            pltpu.VMEM((BPW,), jnp.float32),
            pltpu.SemaphoreType.DMA,
        ],
        compiler_params=pltpu.CompilerParams(needs_layout_passes=False),
    )(_dots_body)


def _loss_body(up_ref, un_ref, o_ref):
    d = jax.nn.sigmoid(up_ref[...]) - jax.nn.sigmoid(un_ref[...])
    o_ref[0, 0] = jnp.sum(-jax.nn.log_sigmoid(d)) * (1.0 / BATCH_SIZE)


_tc_loss = pl.pallas_call(
    _loss_body,
    out_specs=pl.BlockSpec(memory_space=pltpu.SMEM),
    out_shape=jax.ShapeDtypeStruct((1, 1), jnp.float32),
)


def kernel(user_indices, pos_item_indices, neg_item_indices,
           user_embedding, item_embedding):
    idx = jnp.stack([user_indices, pos_item_indices, neg_item_indices])
    idx = idx.astype(jnp.int32)
    jt = (idx >> 1).reshape(3, NUM_WORKERS, N_CHUNKS, CHUNK).transpose(1, 0, 2, 3)
    hv = (idx & 1).reshape(3, NUM_WORKERS, N_CHUNKS, CHUNK).transpose(1, 0, 2, 3)
    u2 = user_embedding.reshape(-1, 2 * DIM)
    i2 = item_embedding.reshape(-1, 2 * DIM)
    up, un = _sc_dots()(jt, hv, u2, i2)
    out = _tc_loss(up.reshape(128, 128), un.reshape(128, 128))
    return out[0, 0]
